# baseline (device time: 12976 ns/iter reference)
import jax
import jax.numpy as jnp
from jax import lax
from jax.experimental import pallas as pl
from jax.experimental.pallas import tpu as pltpu

N_DEV = 4


def kernel(x, w_mat):
    m, k = x.shape
    n = w_mat.shape[1]
    nblk = n // N_DEV

    def body(x_ref, w_ref, out_ref, y_ref, recv_ref, send_sems, recv_sems):
        my = lax.axis_index("i")

        barrier = pltpu.get_barrier_semaphore()
        for d in (1, 2, 3):
            pl.semaphore_signal(
                barrier, inc=1,
                device_id=((my + d) % N_DEV,),
                device_id_type=pl.DeviceIdType.MESH,
            )
        pl.semaphore_wait(barrier, N_DEV - 1)

        x_bf = x_ref[:, :].astype(jnp.bfloat16)
        w_bf = w_ref[:, :].astype(jnp.bfloat16)
        y = jnp.dot(x_bf, w_bf, preferred_element_type=jnp.float32)
        y_ref[:, :] = jnp.maximum(y, 0.0).astype(jnp.bfloat16)

        rdmas = []
        for d in (1, 2, 3):
            tgt = (my + d) % N_DEV
            rdma = pltpu.make_async_remote_copy(
                src_ref=y_ref.at[:, pl.ds(tgt * nblk, nblk)],
                dst_ref=recv_ref.at[d - 1],
                send_sem=send_sems.at[d - 1],
                recv_sem=recv_sems.at[d - 1],
                device_id=(tgt,),
                device_id_type=pl.DeviceIdType.MESH,
            )
            rdma.start()
            rdmas.append(rdma)

        out_ref[pl.ds(my * m, m), :] = y_ref[:, pl.ds(my * nblk, nblk)].astype(
            jnp.float32
        )

        for d in (1, 2, 3):
            rdmas[d - 1].wait()
            src = (my - d) % N_DEV
            out_ref[pl.ds(src * m, m), :] = recv_ref[d - 1, :, :].astype(
                jnp.float32
            )

    return pl.pallas_call(
        body,
        out_shape=jax.ShapeDtypeStruct((n, nblk), jnp.float32),
        in_specs=[
            pl.BlockSpec(memory_space=pltpu.VMEM),
            pl.BlockSpec(memory_space=pltpu.VMEM),
        ],
        out_specs=pl.BlockSpec(memory_space=pltpu.VMEM),
        scratch_shapes=[
            pltpu.VMEM((m, n), jnp.bfloat16),
            pltpu.VMEM((N_DEV - 1, m, nblk), jnp.bfloat16),
            pltpu.SemaphoreType.DMA((N_DEV - 1,)),
            pltpu.SemaphoreType.DMA((N_DEV - 1,)),
        ],
        compiler_params=pltpu.CompilerParams(collective_id=0),
    )(x, w_mat)


# device time: 12725 ns/iter; 1.0197x vs baseline; 1.0197x over previous
import jax
import jax.numpy as jnp
from jax import lax
from jax.experimental import pallas as pl
from jax.experimental.pallas import tpu as pltpu

N_DEV = 4

_SEND_ORDER = (2, 1, 3)
_WAIT_ORDER = (1, 3, 2)


def kernel(x, w_mat):
    m, k = x.shape
    n = w_mat.shape[1]
    nblk = n // N_DEV

    def body(x_ref, w_ref, out_ref, stage_ref, send_ref, send_sems, recv_sems):
        my = lax.axis_index("i")

        barrier = pltpu.get_barrier_semaphore()
        for d in (1, 2, 3):
            pl.semaphore_signal(
                barrier, inc=1,
                device_id=((my + d) % N_DEV,),
                device_id_type=pl.DeviceIdType.MESH,
            )
        pl.semaphore_wait(barrier, N_DEV - 1)

        x_bf = x_ref[:, :].astype(jnp.bfloat16)

        rdmas = []
        for d in _SEND_ORDER:
            tgt = (my + d) % N_DEV
            w_bf = w_ref[:, pl.ds(tgt * nblk, nblk)].astype(jnp.bfloat16)
            blk = jnp.dot(x_bf, w_bf, preferred_element_type=jnp.float32)
            send_ref[d - 1] = jnp.maximum(blk, 0.0).astype(jnp.bfloat16)
            rdma = pltpu.make_async_remote_copy(
                src_ref=send_ref.at[d - 1],
                dst_ref=stage_ref.at[pl.ds(my * m, m), :],
                send_sem=send_sems.at[d - 1],
                recv_sem=recv_sems.at[d - 1],
                device_id=(tgt,),
                device_id_type=pl.DeviceIdType.MESH,
            )
            rdma.start()
            rdmas.append((d, rdma))

        w_bf = w_ref[:, pl.ds(my * nblk, nblk)].astype(jnp.bfloat16)
        blk = jnp.dot(x_bf, w_bf, preferred_element_type=jnp.float32)
        stage_ref[pl.ds(my * m, m), :] = jnp.maximum(blk, 0.0).astype(
            jnp.bfloat16
        )

        waits = dict(rdmas)
        for d in _WAIT_ORDER:
            waits[d].wait()
        out_ref[:, :] = stage_ref[:, :].astype(jnp.float32)

    return pl.pallas_call(
        body,
        out_shape=jax.ShapeDtypeStruct((n, nblk), jnp.float32),
        in_specs=[
            pl.BlockSpec(memory_space=pltpu.VMEM),
            pl.BlockSpec(memory_space=pltpu.VMEM),
        ],
        out_specs=pl.BlockSpec(memory_space=pltpu.VMEM),
        scratch_shapes=[
            pltpu.VMEM((n, nblk), jnp.bfloat16),
            pltpu.VMEM((N_DEV - 1, m, nblk), jnp.bfloat16),
            pltpu.SemaphoreType.DMA((N_DEV - 1,)),
            pltpu.SemaphoreType.DMA((N_DEV - 1,)),
        ],
        compiler_params=pltpu.CompilerParams(collective_id=0),
    )(x, w_mat)
